# baseline (device time: 23172 ns/iter reference)
import jax
import jax.numpy as jnp
from jax import lax
from jax.experimental import pallas as pl
from jax.experimental.pallas import tpu as pltpu

N_DEV = 16
M = 512
N = 512
CHUNK = M // N_DEV

_OFFSETS = sorted(range(1, N_DEV), key=lambda off: min(off, N_DEV - off))


def kernel(x):
    def body(
        x_ref,
        out_ref,
        xbf,
        ag_src,
        rs_buf,
        ag_buf,
        ready_sems,
        rs_send_sems,
        rs_recv_sems,
        ag_send_sems,
        ag_recv_sems,
    ):
        my = lax.axis_index("i")

        barrier = pltpu.get_barrier_semaphore()
        pl.semaphore_signal(
            barrier, inc=1, device_id=(my,),
            device_id_type=pl.DeviceIdType.MESH,
        )

        for off in _OFFSETS:
            pl.semaphore_signal(
                ready_sems.at[N_DEV - off], inc=1,
                device_id=((my + off) % N_DEV,),
                device_id_type=pl.DeviceIdType.MESH,
            )

        xbf[:, :] = x_ref[:, :].astype(jnp.bfloat16)
        pl.semaphore_wait(barrier, 1)

        rs = {}
        for off in _OFFSETS:
            s = N_DEV - off
            tgt = (my + off) % N_DEV
            d = pltpu.make_async_remote_copy(
                src_ref=xbf.at[pl.ds(tgt * CHUNK, CHUNK), :],
                dst_ref=rs_buf.at[s],
                send_sem=rs_send_sems.at[s],
                recv_sem=rs_recv_sems.at[s],
                device_id=(tgt,),
                device_id_type=pl.DeviceIdType.MESH,
            )
            pl.semaphore_wait(ready_sems.at[off], 1)
            d.start()
            rs[s] = d

        reduced = x_ref[pl.ds(my * CHUNK, CHUNK), :].astype(jnp.bfloat16)
        for off in _OFFSETS:
            s = N_DEV - off
            d = rs[s]
            d.wait_recv()
            reduced = reduced + rs_buf[s, :, :]

        ag_src[:, :] = reduced

        ag = {}
        for off in _OFFSETS:
            s = N_DEV - off
            tgt = (my + off) % N_DEV
            d = pltpu.make_async_remote_copy(
                src_ref=ag_src,
                dst_ref=ag_buf.at[s],
                send_sem=ag_send_sems.at[s],
                recv_sem=ag_recv_sems.at[s],
                device_id=(tgt,),
                device_id_type=pl.DeviceIdType.MESH,
            )
            d.start()
            ag[s] = d

        out_ref[pl.ds(my * CHUNK, CHUNK), :] = reduced.astype(jnp.float32)

        for off in _OFFSETS:
            s = N_DEV - off
            d = ag[s]
            d.wait_recv()
            row = ((my + s) % N_DEV) * CHUNK
            out_ref[pl.ds(row, CHUNK), :] = ag_buf[s, :, :].astype(jnp.float32)

        for d in rs.values():
            d.wait_send()
        for d in ag.values():
            d.wait_send()

    return pl.pallas_call(
        body,
        out_shape=jax.ShapeDtypeStruct((M, N), jnp.float32),
        in_specs=[pl.BlockSpec(memory_space=pltpu.VMEM)],
        out_specs=pl.BlockSpec(memory_space=pltpu.VMEM),
        scratch_shapes=[
            pltpu.VMEM((M, N), jnp.bfloat16),
            pltpu.VMEM((CHUNK, N), jnp.bfloat16),
            pltpu.VMEM((N_DEV, CHUNK, N), jnp.bfloat16),
            pltpu.VMEM((N_DEV, CHUNK, N), jnp.bfloat16),
            pltpu.SemaphoreType.REGULAR((N_DEV,)),
            pltpu.SemaphoreType.DMA((N_DEV,)),
            pltpu.SemaphoreType.DMA((N_DEV,)),
            pltpu.SemaphoreType.DMA((N_DEV,)),
            pltpu.SemaphoreType.DMA((N_DEV,)),
        ],
        compiler_params=pltpu.CompilerParams(collective_id=0),
    )(x)


# device time: 21822 ns/iter; 1.0619x vs baseline; 1.0619x over previous
import jax
import jax.numpy as jnp
from jax import lax
from jax.experimental import pallas as pl
from jax.experimental.pallas import tpu as pltpu

N_DEV = 16
M = 512
N = 512
CHUNK = M // N_DEV

_OFFSETS = sorted(range(1, N_DEV), key=lambda off: min(off, N_DEV - off))


def kernel(x):
    def body(
        x_ref,
        out_ref,
        xbf,
        ag_src,
        rs_buf,
        ag_buf,
        rs_send_sems,
        rs_recv_sems,
        ag_send_sems,
        ag_recv_sems,
    ):
        my = lax.axis_index("i")

        barrier = pltpu.get_barrier_semaphore()
        for off in _OFFSETS:
            pl.semaphore_signal(
                barrier, inc=1,
                device_id=((my + off) % N_DEV,),
                device_id_type=pl.DeviceIdType.MESH,
            )
        xbf[:, :] = x_ref[:, :].astype(jnp.bfloat16)
        pl.semaphore_wait(barrier, N_DEV - 1)

        rs = {}
        for off in _OFFSETS:
            s = N_DEV - off
            tgt = (my + off) % N_DEV
            d = pltpu.make_async_remote_copy(
                src_ref=xbf.at[pl.ds(tgt * CHUNK, CHUNK), :],
                dst_ref=rs_buf.at[s],
                send_sem=rs_send_sems.at[s],
                recv_sem=rs_recv_sems.at[s],
                device_id=(tgt,),
                device_id_type=pl.DeviceIdType.MESH,
            )
            d.start()
            rs[s] = d

        reduced = x_ref[pl.ds(my * CHUNK, CHUNK), :].astype(jnp.bfloat16)
        for off in _OFFSETS:
            s = N_DEV - off
            rs[s].wait_recv()
            reduced = reduced + rs_buf[s, :, :]

        ag_src[:, :] = reduced

        ag = {}
        for off in _OFFSETS:
            s = N_DEV - off
            tgt = (my + off) % N_DEV
            d = pltpu.make_async_remote_copy(
                src_ref=ag_src,
                dst_ref=ag_buf.at[s],
                send_sem=ag_send_sems.at[s],
                recv_sem=ag_recv_sems.at[s],
                device_id=(tgt,),
                device_id_type=pl.DeviceIdType.MESH,
            )
            d.start()
            ag[s] = d

        out_ref[pl.ds(my * CHUNK, CHUNK), :] = reduced.astype(jnp.float32)

        for off in _OFFSETS:
            s = N_DEV - off
            ag[s].wait_recv()
            row = ((my + s) % N_DEV) * CHUNK
            out_ref[pl.ds(row, CHUNK), :] = ag_buf[s, :, :].astype(jnp.float32)

        for d in rs.values():
            d.wait_send()
        for d in ag.values():
            d.wait_send()

    return pl.pallas_call(
        body,
        out_shape=jax.ShapeDtypeStruct((M, N), jnp.float32),
        in_specs=[pl.BlockSpec(memory_space=pltpu.VMEM)],
        out_specs=pl.BlockSpec(memory_space=pltpu.VMEM),
        scratch_shapes=[
            pltpu.VMEM((M, N), jnp.bfloat16),
            pltpu.VMEM((CHUNK, N), jnp.bfloat16),
            pltpu.VMEM((N_DEV, CHUNK, N), jnp.bfloat16),
            pltpu.VMEM((N_DEV, CHUNK, N), jnp.bfloat16),
            pltpu.SemaphoreType.DMA((N_DEV,)),
            pltpu.SemaphoreType.DMA((N_DEV,)),
            pltpu.SemaphoreType.DMA((N_DEV,)),
            pltpu.SemaphoreType.DMA((N_DEV,)),
        ],
        compiler_params=pltpu.CompilerParams(collective_id=0),
    )(x)
